# baseline (speedup 1.0000x reference)
import jax
import jax.numpy as jnp
from jax import lax
from jax.experimental import pallas as pl
from jax.experimental.pallas import tpu as pltpu

TOP_K = 2
AUX_COEF = 0.01
TM = 1024


def _router_body(x_ref, wt_ref, rw_hbm, sel_hbm, logits_hbm, aux_ref,
                 acc_ref, srw_ref, ssel_ref, slog_ref, sem_rw, sem_sel, sem_log):
    i = pl.program_id(0)
    nsteps = pl.num_programs(0)
    E = wt_ref.shape[1]
    tm = x_ref.shape[0]
    T_total = tm * nsteps
    d = lax.rem(i, 2)

    def out_dmas(dd, step_idx):
        rows = pl.ds(step_idx * tm, tm)
        return (
            pltpu.make_async_copy(srw_ref.at[dd], rw_hbm.at[rows, :], sem_rw.at[dd]),
            pltpu.make_async_copy(ssel_ref.at[dd], sel_hbm.at[rows, :], sem_sel.at[dd]),
            pltpu.make_async_copy(slog_ref.at[dd], logits_hbm.at[rows, :], sem_log.at[dd]),
        )

    @pl.when(i >= 2)
    def _drain():
        for c in out_dmas(d, i - 2):
            c.wait()

    logits = jnp.dot(x_ref[...], wt_ref[...], preferred_element_type=jnp.float32)

    m = jnp.max(logits, axis=-1, keepdims=True)
    e = jnp.exp(logits - m)
    s = jnp.sum(e, axis=-1, keepdims=True)
    p = e / s

    iota = lax.broadcasted_iota(jnp.int32, (tm, E), 1)
    idx1 = jnp.min(jnp.where(logits == m, iota, E), axis=-1, keepdims=True)
    l2 = jnp.where(iota == idx1, -jnp.inf, logits)
    m2 = jnp.max(l2, axis=-1, keepdims=True)
    idx2 = jnp.min(jnp.where(l2 == m2, iota, E), axis=-1, keepdims=True)

    p1 = jnp.sum(jnp.where(iota == idx1, p, 0.0), axis=-1, keepdims=True)
    p2 = jnp.sum(jnp.where(iota == idx2, p, 0.0), axis=-1, keepdims=True)
    denom = p1 + p2

    slog_ref[d] = logits
    srw_ref[d] = jnp.concatenate([p1 / denom, p2 / denom], axis=1)
    ssel_ref[d] = jnp.concatenate([idx1, idx2], axis=1)
    for c in out_dmas(d, i):
        c.start()

    f_part = jnp.sum(jnp.where(iota == idx1, 1.0, 0.0), axis=0, keepdims=True)
    p_part = jnp.sum(p, axis=0, keepdims=True)

    @pl.when(i == 0)
    def _init():
        acc_ref[...] = jnp.zeros_like(acc_ref)

    acc_ref[0:1, :] += f_part
    acc_ref[1:2, :] += p_part

    @pl.when(i == nsteps - 1)
    def _finish():
        for c in out_dmas(1 - d, i - 1):
            c.wait()
        for c in out_dmas(d, i):
            c.wait()
        aux = (AUX_COEF * E / (float(T_total) * float(T_total))) * jnp.sum(
            acc_ref[0:1, :] * acc_ref[1:2, :]
        )
        aux_ref[...] = jnp.reshape(aux, (1, 1))


def kernel(hidden_states, W):
    T, H = hidden_states.shape
    E = W.shape[0]
    wt = W.T
    grid = (T // TM,)
    rw, sel, logits, aux = pl.pallas_call(
        _router_body,
        grid=grid,
        in_specs=[
            pl.BlockSpec((TM, H), lambda i: (i, 0)),
            pl.BlockSpec((H, E), lambda i: (0, 0)),
        ],
        out_specs=[
            pl.BlockSpec(memory_space=pl.ANY),
            pl.BlockSpec(memory_space=pl.ANY),
            pl.BlockSpec(memory_space=pl.ANY),
            pl.BlockSpec((1, 1), lambda i: (0, 0)),
        ],
        out_shape=[
            jax.ShapeDtypeStruct((T, TOP_K), jnp.float32),
            jax.ShapeDtypeStruct((T, TOP_K), jnp.int32),
            jax.ShapeDtypeStruct((T, E), jnp.float32),
            jax.ShapeDtypeStruct((1, 1), jnp.float32),
        ],
        scratch_shapes=[
            pltpu.VMEM((2, E), jnp.float32),
            pltpu.VMEM((2, TM, TOP_K), jnp.float32),
            pltpu.VMEM((2, TM, TOP_K), jnp.int32),
            pltpu.VMEM((2, TM, E), jnp.float32),
            pltpu.SemaphoreType.DMA((2,)),
            pltpu.SemaphoreType.DMA((2,)),
            pltpu.SemaphoreType.DMA((2,)),
        ],
    )(hidden_states, wt)
    return rw, sel, logits, aux[0, 0]
